# Initial kernel scaffold; baseline (speedup 1.0000x reference)
import jax
import jax.numpy as jnp
from jax.experimental import pallas as pl


def kernel(features, indices):
    raise NotImplementedError("write your pallas kernel here")


# speedup vs baseline: 17.3886x; 17.3886x over previous
import functools

import jax
import jax.numpy as jnp
from jax import lax
from jax.experimental import pallas as pl
from jax.experimental.pallas import tpu as pltpu
from jax.experimental.pallas import tpu_sc as plsc

B = 8
H = W = 112
C = 96
F = H * W * C
UPS = 2
S = (H * UPS) * (W * UPS) * C

NC = 2
NS = 16
L = 16

NCHUNK = 3
CHUNK = S // NCHUNK
DUMP = 2048
ACC = CHUNK + DUMP
NTASK = B * NCHUNK

PER_TILE = F // NS
PIECE = 6272
NPIECE = PER_TILE // PIECE
ROWS = PIECE // 128
ACC_Z = ACC // NS
ZW = ACC_Z // 40
OUT_T = CHUNK // NS


def _body(feat_hbm, idx_hbm, out_hbm, idx_v, feat_v, adj_v, zero_v, acc_sh):
    core = lax.axis_index("c")
    tile = lax.axis_index("s")

    def _zinit(g, _):
        zero_v[pl.ds(g * L, L)] = jnp.zeros((L,), jnp.float32)
        return 0

    lax.fori_loop(0, ZW // L, _zinit, 0)

    def _task(i, _):
        t = i * NC + core
        b = t // NCHUNK
        k = t % NCHUNK
        base_k = k * CHUNK
        in_base = b * F + tile * PER_TILE
        for q in range(ACC_Z // ZW):
            pltpu.sync_copy(zero_v, acc_sh.at[pl.ds(tile * ACC_Z + q * ZW, ZW)])
        plsc.subcore_barrier()

        def _piece(p, _):
            src = in_base + p * PIECE
            pltpu.sync_copy(idx_hbm.at[pl.ds(src, PIECE)], idx_v)
            pltpu.sync_copy(feat_hbm.at[pl.ds(src, PIECE)], feat_v)

            def _row_adjust(j, _):
                for g in range(128 // L):
                    raw = idx_v[pl.ds(j * 128 + g * L, L)]
                    local = raw - base_k
                    ok = (local >= 0) & (local < CHUNK)
                    dump = CHUNK + (raw & (DUMP - 1))
                    adj_v[j, pl.ds(g * L, L)] = jnp.where(ok, local, dump)
                return 0

            lax.fori_loop(0, ROWS, _row_adjust, 0)

            def _row_scatter(j, _):
                pltpu.sync_copy(feat_v.at[pl.ds(j * 128, 128)],
                                acc_sh.at[adj_v.at[j]], add=True)
                return 0

            lax.fori_loop(0, ROWS, _row_scatter, 0)
            return 0

        lax.fori_loop(0, NPIECE, _piece, 0)
        plsc.subcore_barrier()

        out_base = b * S + base_k + tile * OUT_T
        pltpu.sync_copy(acc_sh.at[pl.ds(tile * OUT_T, OUT_T)],
                        out_hbm.at[pl.ds(out_base, OUT_T)])
        plsc.subcore_barrier()
        return 0

    lax.fori_loop(0, NTASK // NC, _task, 0)


@functools.partial(
    pl.kernel,
    out_type=jax.ShapeDtypeStruct((B * S,), jnp.float32),
    mesh=plsc.VectorSubcoreMesh(core_axis_name="c", subcore_axis_name="s"),
    scratch_types=[
        pltpu.VMEM((PIECE,), jnp.int32),
        pltpu.VMEM((PIECE,), jnp.float32),
        pltpu.VMEM((ROWS, 128), jnp.int32),
        pltpu.VMEM((ZW,), jnp.float32),
        pltpu.VMEM_SHARED((ACC,), jnp.float32),
    ],
)
def _scatter_add_kernel(feat_hbm, idx_hbm, out_hbm, idx_v, feat_v, adj_v,
                        zero_v, acc_sh):
    _body(feat_hbm, idx_hbm, out_hbm, idx_v, feat_v, adj_v, zero_v, acc_sh)


def kernel(features, indices):
    feat_flat = features.reshape(B * F)
    idx_flat = indices.reshape(B * F).astype(jnp.int32)
    out = _scatter_add_kernel(feat_flat, idx_flat)
    return out.reshape(B, H * UPS, W * UPS, C)


# speedup vs baseline: 21.9603x; 1.2629x over previous
import functools

import jax
import jax.numpy as jnp
from jax import lax
from jax.experimental import pallas as pl
from jax.experimental.pallas import tpu as pltpu
from jax.experimental.pallas import tpu_sc as plsc

B = 8
H = W = 112
C = 96
F = H * W * C
UPS = 2
S = (H * UPS) * (W * UPS) * C

NC = 2
NS = 16
L = 16

NCHUNK = 3
CHUNK = S // NCHUNK
DUMP = 2048
ACC = CHUNK + DUMP
NTASK = B * NCHUNK

PER_TILE = F // NS
PIECE = 3584
NPIECE = PER_TILE // PIECE
ROWS = PIECE // 128
ACC_Z = ACC // NS
ZW = ACC_Z // 40
OUT_T = CHUNK // NS


def _body(feat_hbm, idx_hbm, out_hbm, idx_v, feat_v, adj_v, zero_v, acc_sh,
          sem_in, sem_sc, sem_z):
    core = lax.axis_index("c")
    tile = lax.axis_index("s")

    def _zinit(g, _):
        zero_v[pl.ds(g * L, L)] = jnp.zeros((L,), jnp.float32)
        return 0

    lax.fori_loop(0, ZW // L, _zinit, 0)

    def _stage_start(p, par, in_base):
        src = in_base + p * PIECE
        pltpu.async_copy(idx_hbm.at[pl.ds(src, PIECE)],
                         idx_v.at[pl.ds(par * PIECE, PIECE)], sem_in)
        pltpu.async_copy(feat_hbm.at[pl.ds(src, PIECE)],
                         feat_v.at[pl.ds(par * PIECE, PIECE)], sem_in)

    def _stage_wait(p, par, in_base):
        src = in_base + p * PIECE
        pltpu.make_async_copy(idx_hbm.at[pl.ds(src, PIECE)],
                              idx_v.at[pl.ds(par * PIECE, PIECE)],
                              sem_in).wait()
        pltpu.make_async_copy(feat_hbm.at[pl.ds(src, PIECE)],
                              feat_v.at[pl.ds(par * PIECE, PIECE)],
                              sem_in).wait()

    def _scatter_fire(par):
        def _row(j, _):
            pltpu.async_copy(feat_v.at[pl.ds(par * PIECE + j * 128, 128)],
                             acc_sh.at[adj_v.at[par * ROWS + j]], sem_sc,
                             add=True)
            return 0

        lax.fori_loop(0, ROWS, _row, 0)

    def _scatter_drain(par):
        def _row(j, _):
            pltpu.make_async_copy(
                feat_v.at[pl.ds(par * PIECE + j * 128, 128)],
                acc_sh.at[adj_v.at[par * ROWS + j]], sem_sc).wait()
            return 0

        lax.fori_loop(0, ROWS, _row, 0)

    def _adjust(par, base_k):
        def _row(j, _):
            for g in range(128 // L):
                raw = idx_v[pl.ds(par * PIECE + j * 128 + g * L, L)]
                local = raw - base_k
                ok = (local >= 0) & (local < CHUNK)
                dump = CHUNK + (raw & (DUMP - 1))
                adj_v[par * ROWS + j, pl.ds(g * L, L)] = jnp.where(ok, local,
                                                                   dump)
            return 0

        lax.fori_loop(0, ROWS, _row, 0)

    def _task(i, _):
        t = i * NC + core
        b = t // NCHUNK
        k = t % NCHUNK
        base_k = k * CHUNK
        in_base = b * F + tile * PER_TILE

        def _zfire(q, _):
            pltpu.async_copy(zero_v,
                             acc_sh.at[pl.ds(tile * ACC_Z + q * ZW, ZW)],
                             sem_z)
            return 0

        def _zdrain(q, _):
            pltpu.make_async_copy(
                zero_v, acc_sh.at[pl.ds(tile * ACC_Z + q * ZW, ZW)],
                sem_z).wait()
            return 0

        lax.fori_loop(0, ACC_Z // ZW, _zfire, 0)
        lax.fori_loop(0, ACC_Z // ZW, _zdrain, 0)
        plsc.subcore_barrier()

        _stage_start(0, 0, in_base)

        def _piece(p, _):
            cur = lax.rem(p, 2)
            nxt = 1 - cur
            _stage_wait(p, cur, in_base)
            _adjust(cur, base_k)

            @pl.when(p > 0)
            def _():
                _scatter_drain(nxt)

            _scatter_fire(cur)

            @pl.when(p + 1 < NPIECE)
            def _():
                _stage_start(p + 1, nxt, in_base)

            return 0

        lax.fori_loop(0, NPIECE, _piece, 0)
        _scatter_drain((NPIECE - 1) % 2)
        plsc.subcore_barrier()

        out_base = b * S + base_k + tile * OUT_T
        pltpu.sync_copy(acc_sh.at[pl.ds(tile * OUT_T, OUT_T)],
                        out_hbm.at[pl.ds(out_base, OUT_T)])
        plsc.subcore_barrier()
        return 0

    lax.fori_loop(0, NTASK // NC, _task, 0)


@functools.partial(
    pl.kernel,
    out_type=jax.ShapeDtypeStruct((B * S,), jnp.float32),
    mesh=plsc.VectorSubcoreMesh(core_axis_name="c", subcore_axis_name="s"),
    scratch_types=[
        pltpu.VMEM((2 * PIECE,), jnp.int32),
        pltpu.VMEM((2 * PIECE,), jnp.float32),
        pltpu.VMEM((2 * ROWS, 128), jnp.int32),
        pltpu.VMEM((ZW,), jnp.float32),
        pltpu.VMEM_SHARED((ACC,), jnp.float32),
        pltpu.SemaphoreType.DMA,
        pltpu.SemaphoreType.DMA,
        pltpu.SemaphoreType.DMA,
    ],
)
def _scatter_add_kernel(feat_hbm, idx_hbm, out_hbm, idx_v, feat_v, adj_v,
                        zero_v, acc_sh, sem_in, sem_sc, sem_z):
    _body(feat_hbm, idx_hbm, out_hbm, idx_v, feat_v, adj_v, zero_v, acc_sh,
          sem_in, sem_sc, sem_z)


def kernel(features, indices):
    feat_flat = features.reshape(B * F)
    idx_flat = indices.reshape(B * F).astype(jnp.int32)
    out = _scatter_add_kernel(feat_flat, idx_flat)
    return out.reshape(B, H * UPS, W * UPS, C)


# speedup vs baseline: 23.2997x; 1.0610x over previous
import functools

import jax
import jax.numpy as jnp
from jax import lax
from jax.experimental import pallas as pl
from jax.experimental.pallas import tpu as pltpu
from jax.experimental.pallas import tpu_sc as plsc

B = 8
H = W = 112
C = 96
F = H * W * C
UPS = 2
OH = H * UPS
OW = W * UPS
PLANE = OW * C
S = OH * PLANE

NC = 2
NS = 16
L = 16

NCHUNK = 3
KP = 75
CHUNKA = KP * PLANE
CHUNKB = (OH - 2 * KP) * PLANE
DUMP = 2048
ACC = CHUNKA + DUMP
NTASK = B * NCHUNK

PER_TILE = F // NS
PIECE = 3584
NPIECE = PER_TILE // PIECE
ROWS = PIECE // 128
ACC_Z = ACC // NS
ZW = 1328
WB_A0 = 788 * 128
WB_A1 = 787 * 128
OUT_TB = CHUNKB // NS


def _body(feat_hbm, idx_hbm, out_hbm, idx_v, feat_v, adj_v, zero_v, acc_sh,
          sem_in, sem_sc, sem_z):
    core = lax.axis_index("c")
    tile = lax.axis_index("s")

    def _zinit(g, _):
        zero_v[pl.ds(g * L, L)] = jnp.zeros((L,), jnp.float32)
        return 0

    lax.fori_loop(0, ZW // L, _zinit, 0)

    def _stage_start(p, par, in_base):
        src = in_base + p * PIECE
        pltpu.async_copy(idx_hbm.at[pl.ds(src, PIECE)],
                         idx_v.at[pl.ds(par * PIECE, PIECE)], sem_in)
        pltpu.async_copy(feat_hbm.at[pl.ds(src, PIECE)],
                         feat_v.at[pl.ds(par * PIECE, PIECE)], sem_in)

    def _stage_wait(p, par, in_base):
        src = in_base + p * PIECE
        pltpu.make_async_copy(idx_hbm.at[pl.ds(src, PIECE)],
                              idx_v.at[pl.ds(par * PIECE, PIECE)],
                              sem_in).wait()
        pltpu.make_async_copy(feat_hbm.at[pl.ds(src, PIECE)],
                              feat_v.at[pl.ds(par * PIECE, PIECE)],
                              sem_in).wait()

    def _scatter_fire(par):
        def _row(j, _):
            pltpu.async_copy(
                feat_v.at[pl.ds(par * PIECE + j * 128, 128)],
                acc_sh.at[adj_v.at[pl.ds(par * PIECE + j * 128, 128)]],
                sem_sc, add=True)
            return 0

        lax.fori_loop(0, ROWS, _row, 0)

    def _scatter_drain(par):
        def _row(j, _):
            pltpu.make_async_copy(
                feat_v.at[pl.ds(par * PIECE + j * 128, 128)],
                acc_sh.at[adj_v.at[pl.ds(par * PIECE + j * 128, 128)]],
                sem_sc).wait()
            return 0

        lax.fori_loop(0, ROWS, _row, 0)

    def _adjust(par, base_k, chunk_size):
        def _row(j, _):
            for g in range(128 // L):
                raw = idx_v[pl.ds(par * PIECE + j * 128 + g * L, L)]
                local = raw - base_k
                ok = (local >= 0) & (local < chunk_size)
                dump = CHUNKA + (raw & (DUMP - 1))
                adj_v[pl.ds(par * PIECE + j * 128 + g * L, L)] = jnp.where(
                    ok, local, dump)
            return 0

        lax.fori_loop(0, ROWS, _row, 0)

    def _task(i, _):
        t = i * NC + core
        b = t // NCHUNK
        k = t % NCHUNK
        base_k = k * CHUNKA
        chunk_size = jnp.where(k == 2, CHUNKB, CHUNKA)
        in_base = b * F + tile * PER_TILE

        def _zfire(q, _):
            pltpu.async_copy(zero_v,
                             acc_sh.at[pl.ds(tile * ACC_Z + q * ZW, ZW)],
                             sem_z)
            return 0

        def _zdrain(q, _):
            pltpu.make_async_copy(
                zero_v, acc_sh.at[pl.ds(tile * ACC_Z + q * ZW, ZW)],
                sem_z).wait()
            return 0

        lax.fori_loop(0, ACC_Z // ZW, _zfire, 0)
        lax.fori_loop(0, ACC_Z // ZW, _zdrain, 0)
        plsc.subcore_barrier()

        _stage_start(0, 0, in_base)

        def _piece(p, _):
            cur = lax.rem(p, 2)
            nxt = 1 - cur
            _stage_wait(p, cur, in_base)
            _adjust(cur, base_k, chunk_size)

            @pl.when(p > 0)
            def _():
                _scatter_drain(nxt)

            _scatter_fire(cur)

            @pl.when(p + 1 < NPIECE)
            def _():
                _stage_start(p + 1, nxt, in_base)

            return 0

        lax.fori_loop(0, NPIECE, _piece, 0)
        _scatter_drain((NPIECE - 1) % 2)
        plsc.subcore_barrier()

        @pl.when((k < 2) & (tile < 8))
        def _():
            off = tile * WB_A0
            pltpu.sync_copy(acc_sh.at[pl.ds(off, WB_A0)],
                            out_hbm.at[pl.ds(b * S + base_k + off, WB_A0)])

        @pl.when((k < 2) & (tile >= 8))
        def _():
            off = 8 * WB_A0 + (tile - 8) * WB_A1
            pltpu.sync_copy(acc_sh.at[pl.ds(off, WB_A1)],
                            out_hbm.at[pl.ds(b * S + base_k + off, WB_A1)])

        @pl.when(k == 2)
        def _():
            off = tile * OUT_TB
            pltpu.sync_copy(acc_sh.at[pl.ds(off, OUT_TB)],
                            out_hbm.at[pl.ds(b * S + base_k + off, OUT_TB)])

        plsc.subcore_barrier()
        return 0

    lax.fori_loop(0, NTASK // NC, _task, 0)


@functools.partial(
    pl.kernel,
    out_type=jax.ShapeDtypeStruct((B * S,), jnp.float32),
    mesh=plsc.VectorSubcoreMesh(core_axis_name="c", subcore_axis_name="s"),
    scratch_types=[
        pltpu.VMEM((2 * PIECE,), jnp.int32),
        pltpu.VMEM((2 * PIECE,), jnp.float32),
        pltpu.VMEM((2 * PIECE,), jnp.int32),
        pltpu.VMEM((ZW,), jnp.float32),
        pltpu.VMEM_SHARED((ACC,), jnp.float32),
        pltpu.SemaphoreType.DMA,
        pltpu.SemaphoreType.DMA,
        pltpu.SemaphoreType.DMA,
    ],
)
def _scatter_add_kernel(feat_hbm, idx_hbm, out_hbm, idx_v, feat_v, adj_v,
                        zero_v, acc_sh, sem_in, sem_sc, sem_z):
    _body(feat_hbm, idx_hbm, out_hbm, idx_v, feat_v, adj_v, zero_v, acc_sh,
          sem_in, sem_sc, sem_z)


def kernel(features, indices):
    feat_flat = features.reshape(B * F)
    idx = indices.reshape(B * F).astype(jnp.int32)
    oh = idx // PLANE
    r = idx - oh * PLANE
    ow = r // C
    oc = r - ow * C
    ridx = oh * PLANE + oc * OW + ow
    out = _scatter_add_kernel(feat_flat, ridx)
    return out.reshape(B, OH, C, OW).transpose(0, 1, 3, 2)
